# baseline (speedup 1.0000x reference)
import jax
import jax.numpy as jnp
from jax import lax
from jax.experimental import pallas as pl
from jax.experimental.pallas import tpu as pltpu
from jax.experimental.pallas import tpu_sc as plsc

B = 1024
T = 200
V = 1000
D = 128
DH = D // 2
BG = B // 16
NTG = (T + 15) // 16
TP = NTG * 16
TREM = T - (NTG - 1) * 16


def _sc_body(idx_hbm, tab_hbm, out_hbm, idx_v, tab_v, tbuf):
    h = lax.axis_index("c")
    bg = lax.axis_index("s")
    b0 = bg * BG

    pltpu.sync_copy(tab_hbm.at[h], tab_v)
    pltpu.sync_copy(idx_hbm.at[pl.ds(b0, BG), :], idx_v)

    iota = lax.iota(jnp.int32, 16)
    lastmask = iota < TREM

    def bi_body(bi, carry):
        def emit_group(base, mask):
            idx16 = idx_v[bi, pl.ds(base, 16)]
            rowb = idx16 * DH

            def d_body(d, sidx):
                vals = plsc.load_gather(tab_v, [rowb + d])
                plsc.store_scatter(tbuf, [sidx], vals, mask=mask)
                return sidx + T

            lax.fori_loop(0, DH, d_body, iota + base, unroll=8)

        def tg_body(tg, c):
            emit_group(pl.multiple_of(tg * 16, 16), None)
            return c

        lax.fori_loop(0, NTG - 1, tg_body, 0)
        emit_group((NTG - 1) * 16, lastmask)

        pltpu.sync_copy(tbuf, out_hbm.at[b0 + bi, pl.ds(h * DH * T, DH * T)])
        return carry

    lax.fori_loop(0, BG, bi_body, 0)


def _sc_lookup_t(idx, tab):
    f = pl.kernel(
        _sc_body,
        out_type=jax.ShapeDtypeStruct((B, D * T), jnp.float32),
        mesh=plsc.VectorSubcoreMesh(core_axis_name="c", subcore_axis_name="s"),
        compiler_params=pltpu.CompilerParams(needs_layout_passes=False),
        scratch_types=[
            pltpu.VMEM((BG, TP), jnp.int32),
            pltpu.VMEM((V * DH,), jnp.float32),
            pltpu.VMEM((DH * T,), jnp.float32),
        ],
    )
    return f(idx, tab)


def kernel(inputs, emb_weight):
    idx = jnp.pad(inputs.astype(jnp.int32), ((0, 0), (0, TP - T)))
    tab = jnp.stack(
        [emb_weight[:, :DH].reshape(-1), emb_weight[:, DH:].reshape(-1)]
    )
    return _sc_lookup_t(idx, tab).reshape(B, D, T)


# speedup vs baseline: 1.6346x; 1.6346x over previous
import jax
import jax.numpy as jnp
from jax import lax
from jax.experimental import pallas as pl
from jax.experimental.pallas import tpu as pltpu
from jax.experimental.pallas import tpu_sc as plsc

B = 1024
T = 200
V = 1000
D = 128
DH = D // 2
BG = B // 16
NTG = (T + 15) // 16
TP = NTG * 16
TREM = T - (NTG - 1) * 16


def _sc_body(idx_hbm, tab_hbm, out_hbm, idx_v, tab_v, tbuf0, tbuf1, sem0, sem1):
    h = lax.axis_index("c")
    bg = lax.axis_index("s")
    b0 = bg * BG

    pltpu.sync_copy(tab_hbm.at[h], tab_v)
    pltpu.sync_copy(idx_hbm.at[pl.ds(b0 * TP, BG * TP)], idx_v)

    iota = lax.iota(jnp.int32, 16)
    lastmask = iota < TREM
    masks = [None] * (NTG - 1) + [lastmask]
    tbufs = (tbuf0, tbuf1)
    sems = (sem0, sem1)

    def out_copy(bi, p):
        return pltpu.make_async_copy(
            tbufs[p], out_hbm.at[b0 + bi, pl.ds(h * DH * T, DH * T)], sems[p]
        )

    def fill(bi, tb):
        ibase = pl.multiple_of(bi * TP, 16)
        fidx0 = tuple(
            idx_v[pl.ds(ibase + tg * 16, 16)] * DH for tg in range(NTG)
        )
        sidx0 = tuple(iota + tg * 16 for tg in range(NTG))

        @plsc.parallel_loop(0, DH, carry=(fidx0, sidx0), unroll=2)
        def d_body(d, c):
            fidx, sidx = c
            nf, ns = [], []
            for tg in range(NTG):
                vals = plsc.load_gather(tab_v, [fidx[tg]])
                plsc.store_scatter(tb, [sidx[tg]], vals, mask=masks[tg])
                nf.append(fidx[tg] + 1)
                ns.append(sidx[tg] + T)
            return tuple(nf), tuple(ns)

    def bi2_body(bi2, carry):
        for p in (0, 1):
            bi = bi2 * 2 + p

            @pl.when(bi2 > 0)
            def _():
                out_copy(bi, p).wait()

            fill(bi, tbufs[p])
            out_copy(bi, p).start()
        return carry

    lax.fori_loop(0, BG // 2, bi2_body, 0)
    for p in (0, 1):
        out_copy(BG - 2 + p, p).wait()


def _sc_lookup_t(idx, tab):
    f = pl.kernel(
        _sc_body,
        out_type=jax.ShapeDtypeStruct((B, D * T), jnp.float32),
        mesh=plsc.VectorSubcoreMesh(core_axis_name="c", subcore_axis_name="s"),
        compiler_params=pltpu.CompilerParams(needs_layout_passes=False),
        scratch_types=[
            pltpu.VMEM((BG * TP,), jnp.int32),
            pltpu.VMEM((V * DH,), jnp.float32),
            pltpu.VMEM((DH * T,), jnp.float32),
            pltpu.VMEM((DH * T,), jnp.float32),
            pltpu.SemaphoreType.DMA,
            pltpu.SemaphoreType.DMA,
        ],
    )
    return f(idx, tab)


def kernel(inputs, emb_weight):
    idx = jnp.pad(inputs.astype(jnp.int32), ((0, 0), (0, TP - T))).reshape(-1)
    tab = jnp.stack(
        [emb_weight[:, :DH].reshape(-1), emb_weight[:, DH:].reshape(-1)]
    )
    return _sc_lookup_t(idx, tab).reshape(B, D, T)
